# baseline (speedup 1.0000x reference)
import functools

import jax
import jax.numpy as jnp
from jax import lax
from jax.experimental import pallas as pl
from jax.experimental.pallas import tpu as pltpu, tpu_sc as plsc

_HI = jax.lax.Precision.HIGHEST


def _argmin_body(x_ref, emb_ref, out_ref):
    B, D, O = x_ref.shape
    K = emb_ref.shape[1]
    e = emb_ref[...]
    e2 = jnp.sum(e * e, axis=0, keepdims=True)
    e_aug = jnp.concatenate(
        [-2.0 * e, jnp.ones((1, K), jnp.float32), e2,
         jnp.zeros((6, K), jnp.float32)], axis=0)
    for b in range(B):
        a = x_ref[b]
        x2 = jnp.sum(a * a, axis=0, keepdims=True)
        a_aug = jnp.concatenate(
            [a, x2, jnp.ones((1, O), jnp.float32),
             jnp.zeros((6, O), jnp.float32)], axis=0)
        dist2 = jax.lax.dot_general(
            e_aug, a_aug, (((0,), (0,)), ((), ())),
            precision=_HI, preferred_element_type=jnp.float32)
        subl = lax.broadcasted_iota(jnp.int32, (K, O), 0)
        mn2 = jnp.min(dist2, axis=0, keepdims=True)
        idx1 = jnp.min(jnp.where(dist2 == mn2, subl, K), axis=0)
        d2m = jnp.where(subl == idx1[None, :], jnp.inf, dist2)
        mn2b = jnp.min(d2m, axis=0, keepdims=True)
        idx2 = jnp.min(jnp.where(d2m == mn2b, subl, K), axis=0)
        lo = jnp.minimum(idx1, idx2)
        hi = jnp.maximum(idx1, idx2)
        oh_lo = (subl == lo[None, :]).astype(jnp.float32)
        oh_hi = (subl == hi[None, :]).astype(jnp.float32)
        ec_lo = jax.lax.dot_general(
            e, oh_lo, (((1,), (0,)), ((), ())),
            precision=_HI, preferred_element_type=jnp.float32)
        ec_hi = jax.lax.dot_general(
            e, oh_hi, (((1,), (0,)), ((), ())),
            precision=_HI, preferred_element_type=jnp.float32)
        sq_lo = (a - ec_lo) * (a - ec_lo)
        sq_hi = (a - ec_hi) * (a - ec_hi)
        acc_lo = sq_lo[0:1]
        acc_hi = sq_hi[0:1]
        for d in range(1, D):
            acc_lo = acc_lo + sq_lo[d:d + 1]
            acc_hi = acc_hi + sq_hi[d:d + 1]
        win = jnp.where(jnp.sqrt(acc_hi) < jnp.sqrt(acc_lo),
                        hi[None, :], lo[None, :])
        out_ref[b] = win[0].astype(jnp.int32)


def _nearest_indices(x, emb):
    B, D, O = x.shape
    K = emb.shape[1]
    return pl.pallas_call(
        _argmin_body,
        out_shape=jax.ShapeDtypeStruct((B, O), jnp.int32),
        compiler_params=pltpu.CompilerParams(
            fuse_transposed_lhs_in_matmul=True,
            disable_bounds_checks=True),
    )(x, emb)


def _make_sc_gather(B, D, O, K):
    info = plsc.get_sparse_core_info()
    NC, NS = info.num_cores, info.num_subcores
    NW = NC * NS
    rows = B * D
    rows_per_w = rows // NW
    d_per_w = D // (NW // B)
    chunks = O // 16
    mesh = plsc.VectorSubcoreMesh(core_axis_name="c", subcore_axis_name="s")

    @functools.partial(
        pl.kernel,
        mesh=mesh,
        out_type=jax.ShapeDtypeStruct((rows, O), jnp.float32),
        scratch_types=[
            pltpu.VMEM((d_per_w, K), jnp.float32),
            pltpu.VMEM((1, O), jnp.int32),
            pltpu.VMEM((rows_per_w, O), jnp.float32),
        ],
        compiler_params=pltpu.CompilerParams(
            needs_layout_passes=False,
            disable_bounds_checks=True,
            disable_semaphore_checks=True),
    )
    def gather(emb_hbm, amin_hbm, out_hbm, emb_v, idx_v, out_v):
        wid = lax.axis_index("s") * NC + lax.axis_index("c")
        b = wid // (NW // B)
        dlo = (wid % (NW // B)) * d_per_w
        pltpu.sync_copy(emb_hbm.at[pl.ds(dlo, d_per_w), :], emb_v)
        pltpu.sync_copy(amin_hbm.at[pl.ds(b, 1), :], idx_v)
        rsplat = [jnp.full((16,), r, jnp.int32) for r in range(rows_per_w)]

        def chunk_body(c, _):
            idx = idx_v[0, pl.ds(c * 16, 16)]
            vals = [plsc.load_gather(emb_v, [rsplat[r], idx])
                    for r in range(rows_per_w)]
            for r in range(rows_per_w):
                out_v[r, pl.ds(c * 16, 16)] = vals[r]
            return 0

        lax.fori_loop(0, chunks, chunk_body, 0, unroll=2)
        pltpu.sync_copy(out_v, out_hbm.at[pl.ds(wid * rows_per_w, rows_per_w), :])

    return gather


def kernel(x, emb):
    B, D, O = x.shape
    K = emb.shape[1]
    amin = _nearest_indices(x, emb)
    gather = _make_sc_gather(B, D, O, K)
    res = gather(emb, amin)
    return res.reshape(B, D, O), amin


# speedup vs baseline: 1.1481x; 1.1481x over previous
import functools

import jax
import jax.numpy as jnp
from jax import lax
from jax.experimental import pallas as pl
from jax.experimental.pallas import tpu as pltpu, tpu_sc as plsc

_HI = jax.lax.Precision.HIGHEST


def _argmin_body(x_ref, emb_ref, out_ref):
    B, D, O = x_ref.shape
    K = emb_ref.shape[1]
    e = emb_ref[...]
    e_hi = (e.astype(jnp.bfloat16)).astype(jnp.float32)
    e_mid = ((e - e_hi).astype(jnp.bfloat16)).astype(jnp.float32)
    e_lo = e - e_hi - e_mid
    e2 = jnp.sum(e * e, axis=0, keepdims=True)
    e_aug = jnp.concatenate(
        [-2.0 * e, jnp.ones((1, K), jnp.float32), e2,
         jnp.zeros((6, K), jnp.float32)], axis=0)
    for b in range(B):
        a = x_ref[b]
        x2 = jnp.sum(a * a, axis=0, keepdims=True)
        a_aug = jnp.concatenate(
            [a, x2, jnp.ones((1, O), jnp.float32),
             jnp.zeros((6, O), jnp.float32)], axis=0)
        dist2 = jax.lax.dot_general(
            e_aug, a_aug, (((0,), (0,)), ((), ())),
            precision=_HI, preferred_element_type=jnp.float32)
        subl = lax.broadcasted_iota(jnp.int32, (K, O), 0)
        mn2 = jnp.min(dist2, axis=0, keepdims=True)
        idx1 = jnp.min(jnp.where(dist2 == mn2, subl, K), axis=0)
        d2m = jnp.where(subl == idx1[None, :], jnp.inf, dist2)
        mn2b = jnp.min(d2m, axis=0, keepdims=True)
        idx2 = jnp.min(jnp.where(d2m == mn2b, subl, K), axis=0)
        lo = jnp.minimum(idx1, idx2)
        hi = jnp.maximum(idx1, idx2)
        oh_lo = (subl == lo[None, :]).astype(jnp.float32)
        oh_hi = (subl == hi[None, :]).astype(jnp.float32)

        def sel(oh):
            parts = [jax.lax.dot_general(
                ch, oh, (((1,), (0,)), ((), ())),
                preferred_element_type=jnp.float32)
                for ch in (e_hi, e_mid, e_lo)]
            return (parts[0] + parts[1]) + parts[2]

        ec_lo = sel(oh_lo)
        ec_hi = sel(oh_hi)
        sq_lo = (a - ec_lo) * (a - ec_lo)
        sq_hi = (a - ec_hi) * (a - ec_hi)
        acc_lo = sq_lo[0:1]
        acc_hi = sq_hi[0:1]
        for d in range(1, D):
            acc_lo = acc_lo + sq_lo[d:d + 1]
            acc_hi = acc_hi + sq_hi[d:d + 1]
        win = jnp.where(jnp.sqrt(acc_hi) < jnp.sqrt(acc_lo),
                        hi[None, :], lo[None, :])
        out_ref[b] = win[0].astype(jnp.int32)


def _nearest_indices(x, emb):
    B, D, O = x.shape
    K = emb.shape[1]
    return pl.pallas_call(
        _argmin_body,
        out_shape=jax.ShapeDtypeStruct((B, O), jnp.int32),
        compiler_params=pltpu.CompilerParams(
            fuse_transposed_lhs_in_matmul=True,
            disable_bounds_checks=True),
    )(x, emb)


def _make_sc_gather(B, D, O, K):
    info = plsc.get_sparse_core_info()
    NC, NS = info.num_cores, info.num_subcores
    NW = NC * NS
    rows = B * D
    rows_per_w = rows // NW
    d_per_w = D // (NW // B)
    chunks = O // 16
    mesh = plsc.VectorSubcoreMesh(core_axis_name="c", subcore_axis_name="s")

    @functools.partial(
        pl.kernel,
        mesh=mesh,
        out_type=jax.ShapeDtypeStruct((rows, O), jnp.float32),
        scratch_types=[
            pltpu.VMEM((d_per_w, K), jnp.float32),
            pltpu.VMEM((1, O), jnp.int32),
            pltpu.VMEM((rows_per_w, O), jnp.float32),
        ],
        compiler_params=pltpu.CompilerParams(
            needs_layout_passes=False,
            disable_bounds_checks=True,
            disable_semaphore_checks=True),
    )
    def gather(emb_hbm, amin_hbm, out_hbm, emb_v, idx_v, out_v):
        wid = lax.axis_index("s") * NC + lax.axis_index("c")
        b = wid // (NW // B)
        dlo = (wid % (NW // B)) * d_per_w
        pltpu.sync_copy(emb_hbm.at[pl.ds(dlo, d_per_w), :], emb_v)
        pltpu.sync_copy(amin_hbm.at[pl.ds(b, 1), :], idx_v)
        rsplat = [jnp.full((16,), r, jnp.int32) for r in range(rows_per_w)]

        def chunk_body(c, _):
            idx = idx_v[0, pl.ds(c * 16, 16)]
            vals = [plsc.load_gather(emb_v, [rsplat[r], idx])
                    for r in range(rows_per_w)]
            for r in range(rows_per_w):
                out_v[r, pl.ds(c * 16, 16)] = vals[r]
            return 0

        lax.fori_loop(0, chunks, chunk_body, 0, unroll=2)
        pltpu.sync_copy(out_v, out_hbm.at[pl.ds(wid * rows_per_w, rows_per_w), :])

    return gather


def kernel(x, emb):
    B, D, O = x.shape
    K = emb.shape[1]
    amin = _nearest_indices(x, emb)
    gather = _make_sc_gather(B, D, O, K)
    res = gather(emb, amin)
    return res.reshape(B, D, O), amin
